# baseline (speedup 1.0000x reference)
import functools

import jax
import jax.numpy as jnp
from jax import lax
from jax.experimental import pallas as pl
from jax.experimental.pallas import tpu as pltpu

HEADS = 3
NEG_SLOPE = 0.2

_LHS_T = (((0,), (0,)), ((), ()))
_STD = (((1,), (0,)), ((), ()))


def _leaky_relu(x):
    return jnp.maximum(x, NEG_SLOPE * x)


def _elu(x):
    return jnp.where(x > 0, x, jnp.exp(x) - 1.0)


def _gat_dense_t(ht, a_src_row, a_dst_row):
    d = ht.shape[0]
    as_row = lax.dot_general(a_src_row, ht, _STD,
                             preferred_element_type=jnp.float32)
    ad_row = lax.dot_general(a_dst_row, ht, _STD,
                             preferred_element_type=jnp.float32)
    as_col = as_row.reshape(-1, 1)
    as_max = jnp.max(as_row, axis=1, keepdims=True)
    e = _leaky_relu(as_col + ad_row)
    emax = _leaky_relu(as_max + ad_row)
    ee = jnp.exp(e - emax)
    ht_aug = jnp.concatenate([ht, jnp.ones_like(ht[:1, :])], axis=0)
    agg = lax.dot_general(ht_aug, ee, _STD,
                          preferred_element_type=jnp.float32)
    return agg[:d, :] / (agg[d:d + 1, :] + 1e-16)


def _gnn_kernel(xt_ref, mask_ref, w1_ref, a1s_ref, a1d_ref, w2_hbm,
                a2s_ref, a2d_ref, p1w_hbm, p1b_ref, p2w_hbm, p2b_ref,
                out_ref, mask_out_ref, w2_v, p1w_v, p2w_v, s_w2, s_p1w, s_p2w,
                *, n, jobs, batch):
    c_w2 = pltpu.make_async_copy(w2_hbm, w2_v, s_w2)
    c_p1w = pltpu.make_async_copy(p1w_hbm, p1w_v, s_p1w)
    c_p2w = pltpu.make_async_copy(p2w_hbm, p2w_v, s_p2w)
    c_w2.start()
    c_p1w.start()
    c_p2w.start()

    xt = xt_ref[...]
    head_outs = []
    for h in range(HEADS):
        w = w1_ref[h]
        ht = lax.dot_general(w, xt, _LHS_T,
                             preferred_element_type=jnp.float32)
        a_s = a1s_ref[pl.ds(h, 1), :]
        a_d = a1d_ref[pl.ds(h, 1), :]
        head_outs.append(_gat_dense_t(ht, a_s, a_d))
    h1t = _elu(jnp.concatenate(head_outs, axis=0))

    c_w2.wait()
    h2t_feat = lax.dot_general(w2_v[...], h1t, _STD,
                               preferred_element_type=jnp.float32)
    h2t = _elu(_gat_dense_t(h2t_feat, a2s_ref[...], a2d_ref[...]))

    c_p1w.wait()
    f1t = jnp.maximum(
        lax.dot_general(p1w_v[...], h2t, _LHS_T,
                        preferred_element_type=jnp.float32)
        + p1b_ref[...].reshape(-1, 1), 0.0)
    c_p2w.wait()
    f2t = jnp.maximum(
        lax.dot_general(p2w_v[...], f1t, _LHS_T,
                        preferred_element_type=jnp.float32)
        + p2b_ref[...].reshape(-1, 1), 0.0)

    col_b = lax.broadcasted_iota(jnp.int32, (n, batch), 1)
    row_n = lax.broadcasted_iota(jnp.int32, (n, batch), 0)
    lo = col_b * jobs
    sel = jnp.where((row_n >= lo) & (row_n < lo + jobs), 1.0 / jobs, 0.0)
    feats_t = lax.dot_general(f2t, sel, _STD,
                              preferred_element_type=jnp.float32)
    out_ref[...] = feats_t.T
    mask_out_ref[...] = mask_ref[...]


@jax.jit
def kernel(real_obs, action_mask, W1, a1_src, a1_dst, W2, a2_src, a2_dst,
           P1w, P1b, P2w, P2b):
    B, J, F = real_obs.shape
    N = B * J
    OUT2 = W2.shape[1]
    HID = P2w.shape[1]

    xt = real_obs.reshape(N, F).T
    w2t = W2.T

    vspec = pl.BlockSpec(memory_space=pltpu.VMEM)
    aspec = pl.BlockSpec(memory_space=pl.ANY)
    body = functools.partial(_gnn_kernel, n=N, jobs=J, batch=B)
    feats, mask_out = pl.pallas_call(
        body,
        in_specs=[vspec, vspec, vspec, vspec, vspec, aspec, vspec, vspec,
                  aspec, vspec, aspec, vspec],
        out_shape=(jax.ShapeDtypeStruct((B, HID), jnp.float32),
                   jax.ShapeDtypeStruct((B, J), action_mask.dtype)),
        scratch_shapes=[pltpu.VMEM(w2t.shape, jnp.float32),
                        pltpu.VMEM(P1w.shape, jnp.float32),
                        pltpu.VMEM(P2w.shape, jnp.float32),
                        pltpu.SemaphoreType.DMA, pltpu.SemaphoreType.DMA,
                        pltpu.SemaphoreType.DMA],
    )(xt, action_mask, W1, a1_src, a1_dst, w2t,
      a2_src.reshape(1, -1), a2_dst.reshape(1, -1),
      P1w, P1b.reshape(1, -1), P2w, P2b.reshape(1, -1))
    return feats, mask_out


# speedup vs baseline: 1.1237x; 1.1237x over previous
import functools

import jax
import jax.numpy as jnp
from jax import lax
from jax.experimental import pallas as pl
from jax.experimental.pallas import tpu as pltpu

HEADS = 3
NEG_SLOPE = 0.2

_LHS_T = (((0,), (0,)), ((), ()))
_STD = (((1,), (0,)), ((), ()))


def _leaky_relu(x):
    return jnp.maximum(x, NEG_SLOPE * x)


def _elu(x):
    return jnp.where(x > 0, x, jnp.exp(x) - 1.0)


def _gat_dense_t(ht, a_src_row, a_dst_row):
    d = ht.shape[0]
    as_row = lax.dot_general(a_src_row, ht, _STD,
                             preferred_element_type=jnp.float32)
    ad_row = lax.dot_general(a_dst_row, ht, _STD,
                             preferred_element_type=jnp.float32)
    as_col = as_row.reshape(-1, 1)
    as_max = jnp.max(as_row, axis=1, keepdims=True)
    ht_aug = jnp.concatenate([ht, jnp.ones_like(ht[:1, :])], axis=0)
    n = ht.shape[1]
    aggs = []
    for lo in range(0, n, 256):
        hi = min(lo + 256, n)
        ad_c = ad_row[:, lo:hi]
        e_c = _leaky_relu(as_col + ad_c)
        emax_c = _leaky_relu(as_max + ad_c)
        ee_c = jnp.exp(e_c - emax_c)
        aggs.append(lax.dot_general(ht_aug, ee_c, _STD,
                                    preferred_element_type=jnp.float32))
    agg = jnp.concatenate(aggs, axis=1)
    return agg[:d, :] / (agg[d:d + 1, :] + 1e-16)


def _gnn_kernel(xt_ref, mask_ref, w1_ref, a1s_ref, a1d_ref, w2_hbm,
                a2s_ref, a2d_ref, p1w_hbm, p1b_ref, p2w_hbm, p2b_ref,
                out_ref, mask_out_ref, w2_v, p1w_v, p2w_v, s_w2, s_p1w, s_p2w,
                *, n, jobs, batch):
    c_w2 = pltpu.make_async_copy(w2_hbm, w2_v, s_w2)
    c_p1w = pltpu.make_async_copy(p1w_hbm, p1w_v, s_p1w)
    c_p2w = pltpu.make_async_copy(p2w_hbm, p2w_v, s_p2w)
    c_w2.start()
    c_p1w.start()
    c_p2w.start()

    xt = xt_ref[...]
    head_outs = []
    for h in range(HEADS):
        w = w1_ref[h]
        ht = lax.dot_general(w, xt, _LHS_T,
                             preferred_element_type=jnp.float32)
        a_s = a1s_ref[pl.ds(h, 1), :]
        a_d = a1d_ref[pl.ds(h, 1), :]
        head_outs.append(_gat_dense_t(ht, a_s, a_d))
    h1t = _elu(jnp.concatenate(head_outs, axis=0))

    c_w2.wait()
    h2t_feat = lax.dot_general(w2_v[...], h1t, _STD,
                               preferred_element_type=jnp.float32)
    h2t = _elu(_gat_dense_t(h2t_feat, a2s_ref[...], a2d_ref[...]))

    c_p1w.wait()
    f1t = jnp.maximum(
        lax.dot_general(p1w_v[...], h2t, _LHS_T,
                        preferred_element_type=jnp.float32)
        + p1b_ref[...].reshape(-1, 1), 0.0)
    c_p2w.wait()
    f2t = jnp.maximum(
        lax.dot_general(p2w_v[...], f1t, _STD,
                        preferred_element_type=jnp.float32)
        + p2b_ref[...].reshape(-1, 1), 0.0)

    col_b = lax.broadcasted_iota(jnp.int32, (n, batch), 1)
    row_n = lax.broadcasted_iota(jnp.int32, (n, batch), 0)
    lo = col_b * jobs
    sel = jnp.where((row_n >= lo) & (row_n < lo + jobs), 1.0 / jobs, 0.0)
    feats_t = lax.dot_general(f2t, sel, _STD,
                              preferred_element_type=jnp.float32)
    out_ref[...] = feats_t.T
    mask_out_ref[...] = mask_ref[...]


@jax.jit
def kernel(real_obs, action_mask, W1, a1_src, a1_dst, W2, a2_src, a2_dst,
           P1w, P1b, P2w, P2b):
    B, J, F = real_obs.shape
    N = B * J
    OUT2 = W2.shape[1]
    HID = P2w.shape[1]

    xt = real_obs.reshape(N, F).T
    w2t = W2.T
    p2wt = P2w.T

    vspec = pl.BlockSpec(memory_space=pltpu.VMEM)
    aspec = pl.BlockSpec(memory_space=pl.ANY)
    body = functools.partial(_gnn_kernel, n=N, jobs=J, batch=B)
    feats, mask_out = pl.pallas_call(
        body,
        in_specs=[vspec, vspec, vspec, vspec, vspec, aspec, vspec, vspec,
                  aspec, vspec, aspec, vspec],
        out_shape=(jax.ShapeDtypeStruct((B, HID), jnp.float32),
                   jax.ShapeDtypeStruct((B, J), action_mask.dtype)),
        scratch_shapes=[pltpu.VMEM(w2t.shape, jnp.float32),
                        pltpu.VMEM(P1w.shape, jnp.float32),
                        pltpu.VMEM(p2wt.shape, jnp.float32),
                        pltpu.SemaphoreType.DMA, pltpu.SemaphoreType.DMA,
                        pltpu.SemaphoreType.DMA],
    )(xt, action_mask, W1, a1_src, a1_dst, w2t,
      a2_src.reshape(1, -1), a2_dst.reshape(1, -1),
      P1w, P1b.reshape(1, -1), p2wt, P2b.reshape(1, -1))
    return feats, mask_out
